# baseline (speedup 1.0000x reference)
import functools

import jax
import jax.numpy as jnp
from jax import lax
from jax.experimental import pallas as pl
from jax.experimental.pallas import tpu as pltpu
from jax.experimental.pallas import tpu_sc as plsc

_GRP = 16


@functools.cache
def _build(B, V, D):
    info = plsc.get_sparse_core_info()
    nw = info.num_cores * info.num_subcores
    b_per_w = B // nw
    n_grp = b_per_w // _GRP
    assert B % (nw * _GRP) == 0

    mesh = plsc.VectorSubcoreMesh(core_axis_name="c", subcore_axis_name="s")

    @functools.partial(
        pl.kernel,
        mesh=mesh,
        out_type=jax.ShapeDtypeStruct((D, B), jnp.float32),
        scratch_types=[
            pltpu.VMEM((b_per_w,), jnp.int32),
            pltpu.VMEM((2 * _GRP, D, 128), jnp.float32),
            pltpu.VMEM((D, b_per_w), jnp.float32),
            pltpu.SemaphoreType.DMA,
            pltpu.SemaphoreType.DMA,
        ],
        compiler_params=pltpu.CompilerParams(
            use_tc_tiling_on_sc=True, needs_layout_passes=False
        ),
    )
    def emb(labels_hbm, table_hbm, out_hbm, idx_v, slots, blk_v, sem0, sem1):
        wid = lax.axis_index("s") * info.num_cores + lax.axis_index("c")
        base = wid * b_per_w
        pltpu.sync_copy(labels_hbm.at[pl.ds(base, b_per_w)], idx_v)
        sems = [sem0, sem1]
        rows = lax.iota(jnp.int32, _GRP)

        def drain(parity):
            for t in range(_GRP):
                pltpu.make_async_copy(
                    table_hbm.at[:, pl.ds(0, 128)],
                    slots.at[parity * _GRP + t],
                    sems[parity],
                ).wait()

        def fire(g, parity):
            vec = idx_v[pl.ds(g * _GRP, _GRP)]
            for t in range(_GRP):
                c = vec[t]
                c0 = pl.multiple_of((c >> 7) * 128, 128)
                pltpu.async_copy(
                    table_hbm.at[:, pl.ds(c0, 128)],
                    slots.at[parity * _GRP + t],
                    sems[parity],
                )

        def pull(g, parity):
            vec = idx_v[pl.ds(g * _GRP, _GRP)]
            for t in range(_GRP):
                c = vec[t]
                lane = jnp.full((_GRP,), c & 127, jnp.int32)
                v = plsc.load_gather(slots.at[parity * _GRP + t], [rows, lane])
                col = jnp.full((_GRP,), g * _GRP + t, jnp.int32)
                plsc.store_scatter(blk_v, [rows, col], v)

        fire(0, 0)

        def body(g, carry):
            parity = lax.rem(g, 2)

            @pl.when(parity == 0)
            def _():
                fire(g, 0)
                drain(1)
                pull(g - 1, 1)

            @pl.when(parity == 1)
            def _():
                fire(g, 1)
                drain(0)
                pull(g - 1, 0)

            return carry

        lax.fori_loop(1, n_grp, body, 0)
        last = n_grp - 1
        drain(last % 2)
        pull(last, last % 2)
        pltpu.sync_copy(blk_v, out_hbm.at[:, pl.ds(base, b_per_w)])

    return emb


def kernel(labels, train, embedding_table):
    del train
    (B,) = labels.shape
    V, D = embedding_table.shape
    out_t = _build(B, V, D)(labels.astype(jnp.int32), embedding_table.T)
    return out_t.T
